# baseline (speedup 1.0000x reference)
import jax
import jax.numpy as jnp
from jax.experimental import pallas as pl
from jax.experimental.pallas import tpu as pltpu

B, C, H, W = 8, 96, 384, 384
E = 8
RED = 12
HW = H * W

CBLK = 16
NJ = C // CBLK
NRED = B * NJ


def _body(x_ref, w1t_ref, w2t_ref, b2_ref, out_ref, acc_ref, w_scr):
    g = pl.program_id(0)
    i = g // NJ
    j = g - i * NJ

    for jj in range(NJ):
        @pl.when((g < NRED) & (j == jj))
        def _(jj=jj):
            acc_ref[:, jj * CBLK:(jj + 1) * CBLK] = jnp.sum(
                x_ref[...], axis=(2, 3))

    @pl.when((g < NRED) & (j == NJ - 1))
    def _():
        pooled = acc_ref[...] * (1.0 / HW)
        hidden = jnp.dot(pooled, w1t_ref[...], preferred_element_type=jnp.float32)
        hidden = hidden * jax.nn.sigmoid(hidden)
        logits = jnp.dot(hidden, w2t_ref[...], preferred_element_type=jnp.float32)
        logits = logits + b2_ref[...]
        m = jnp.max(logits, axis=1, keepdims=True)
        p = jnp.exp(logits - m)
        probs = p / jnp.sum(p, axis=1, keepdims=True)
        iota = jax.lax.broadcasted_iota(jnp.int32, (1, E), 1)
        v1 = jnp.max(probs, axis=1, keepdims=True)
        i1 = jnp.min(jnp.where(probs == v1, iota, E), axis=1, keepdims=True)
        m1 = iota == i1
        pr2 = jnp.where(m1, -1.0, probs)
        v2 = jnp.max(pr2, axis=1, keepdims=True)
        i2 = jnp.min(jnp.where(pr2 == v2, iota, E), axis=1, keepdims=True)
        m2 = iota == i2
        s = v1 + v2 + 1e-6
        w = jnp.where(m1, v1 / s, 0.0) + jnp.where(m2, v2 / s, 0.0)
        w_scr[pl.ds(i, 1), :] = w

    @pl.when(g >= NRED)
    def _():
        w_row = w_scr[pl.ds(g - NRED, 1), :]
        out_ref[...] = jnp.broadcast_to(w_row[0, :, None, None], (E, H, W))[None]


def kernel(x, W1, W2, b2):
    return pl.pallas_call(
        _body,
        grid=(NRED + B,),
        in_specs=[
            pl.BlockSpec(
                (1, CBLK, H, W),
                lambda g: (jnp.minimum(g // NJ, B - 1),
                           jnp.where(g < NRED, g - (g // NJ) * NJ, NJ - 1),
                           0, 0),
            ),
            pl.BlockSpec((C, RED), lambda g: (0, 0)),
            pl.BlockSpec((RED, E), lambda g: (0, 0)),
            pl.BlockSpec((1, E), lambda g: (0, 0)),
        ],
        out_specs=pl.BlockSpec(
            (1, E, H, W),
            lambda g: (jnp.maximum(g - NRED, 0), 0, 0, 0),
        ),
        out_shape=jax.ShapeDtypeStruct((B, E, H, W), jnp.float32),
        scratch_shapes=[
            pltpu.VMEM((1, C), jnp.float32),
            pltpu.VMEM((B, E), jnp.float32),
        ],
    )(x, W1.T, W2.T, b2.reshape(1, E))


# speedup vs baseline: 1.0011x; 1.0011x over previous
import jax
import jax.numpy as jnp
from jax.experimental import pallas as pl
from jax.experimental.pallas import tpu as pltpu

B, C, H, W = 8, 96, 384, 384
E = 8
RED = 12
HW = H * W

CBLK = 16
NJ = C // CBLK
NRED = B * NJ


def _body(x_ref, w1t_ref, w2t_ref, b2_ref, out_ref, acc_ref, w_scr):
    g = pl.program_id(0)
    i = g // NJ
    j = g - i * NJ

    for jj in range(NJ):
        @pl.when((g < NRED) & (j == jj))
        def _(jj=jj):
            acc_ref[:, jj * CBLK:(jj + 1) * CBLK] = jnp.sum(
                x_ref[...], axis=(2, 3))

    @pl.when((g < NRED) & (j == NJ - 1))
    def _():
        pooled = acc_ref[...] * (1.0 / HW)
        hidden = jnp.dot(pooled, w1t_ref[...], preferred_element_type=jnp.float32)
        hidden = hidden * jax.nn.sigmoid(hidden)
        logits = jnp.dot(hidden, w2t_ref[...], preferred_element_type=jnp.float32)
        logits = logits + b2_ref[...]
        m = jnp.max(logits, axis=1, keepdims=True)
        p = jnp.exp(logits - m)
        probs = p / jnp.sum(p, axis=1, keepdims=True)
        iota = jax.lax.broadcasted_iota(jnp.int32, (1, E), 1)
        v1 = jnp.max(probs, axis=1, keepdims=True)
        i1 = jnp.min(jnp.where(probs == v1, iota, E), axis=1, keepdims=True)
        m1 = iota == i1
        pr2 = jnp.where(m1, -1.0, probs)
        v2 = jnp.max(pr2, axis=1, keepdims=True)
        i2 = jnp.min(jnp.where(pr2 == v2, iota, E), axis=1, keepdims=True)
        m2 = iota == i2
        s = v1 + v2 + 1e-6
        w = jnp.where(m1, v1 / s, 0.0) + jnp.where(m2, v2 / s, 0.0)
        w_scr[pl.ds(i, 1), :] = w

    @pl.when(g >= NRED)
    def _():
        w_row = w_scr[pl.ds(g - NRED, 1), :]
        for e in range(E):
            out_ref[0, e] = jnp.broadcast_to(w_row[0, e], (H, W))


def kernel(x, W1, W2, b2):
    return pl.pallas_call(
        _body,
        grid=(NRED + B,),
        in_specs=[
            pl.BlockSpec(
                (1, CBLK, H, W),
                lambda g: (jnp.minimum(g // NJ, B - 1),
                           jnp.where(g < NRED, g - (g // NJ) * NJ, NJ - 1),
                           0, 0),
            ),
            pl.BlockSpec((C, RED), lambda g: (0, 0)),
            pl.BlockSpec((RED, E), lambda g: (0, 0)),
            pl.BlockSpec((1, E), lambda g: (0, 0)),
        ],
        out_specs=pl.BlockSpec(
            (1, E, H, W),
            lambda g: (jnp.maximum(g - NRED, 0), 0, 0, 0),
        ),
        out_shape=jax.ShapeDtypeStruct((B, E, H, W), jnp.float32),
        scratch_shapes=[
            pltpu.VMEM((1, C), jnp.float32),
            pltpu.VMEM((B, E), jnp.float32),
        ],
    )(x, W1.T, W2.T, b2.reshape(1, E))


# speedup vs baseline: 1.0062x; 1.0050x over previous
import jax
import jax.numpy as jnp
from jax.experimental import pallas as pl
from jax.experimental.pallas import tpu as pltpu

B, C, H, W = 8, 96, 384, 384
E = 8
RED = 12
HW = H * W

CBLK = 24
NJ = C // CBLK
NRED = B * NJ


def _body(x_ref, w1t_ref, w2t_ref, b2_ref, out_ref, acc_ref, w_scr):
    g = pl.program_id(0)
    i = g // NJ
    j = g - i * NJ

    for jj in range(NJ):
        @pl.when((g < NRED) & (j == jj))
        def _(jj=jj):
            acc_ref[:, jj * CBLK:(jj + 1) * CBLK] = jnp.sum(
                x_ref[...], axis=(2, 3))

    @pl.when((g < NRED) & (j == NJ - 1))
    def _():
        pooled = acc_ref[...] * (1.0 / HW)
        hidden = jnp.dot(pooled, w1t_ref[...], preferred_element_type=jnp.float32)
        hidden = hidden * jax.nn.sigmoid(hidden)
        logits = jnp.dot(hidden, w2t_ref[...], preferred_element_type=jnp.float32)
        logits = logits + b2_ref[...]
        m = jnp.max(logits, axis=1, keepdims=True)
        p = jnp.exp(logits - m)
        probs = p / jnp.sum(p, axis=1, keepdims=True)
        iota = jax.lax.broadcasted_iota(jnp.int32, (1, E), 1)
        v1 = jnp.max(probs, axis=1, keepdims=True)
        i1 = jnp.min(jnp.where(probs == v1, iota, E), axis=1, keepdims=True)
        m1 = iota == i1
        pr2 = jnp.where(m1, -1.0, probs)
        v2 = jnp.max(pr2, axis=1, keepdims=True)
        i2 = jnp.min(jnp.where(pr2 == v2, iota, E), axis=1, keepdims=True)
        m2 = iota == i2
        s = v1 + v2 + 1e-6
        w = jnp.where(m1, v1 / s, 0.0) + jnp.where(m2, v2 / s, 0.0)
        w_scr[pl.ds(i, 1), :] = w

    @pl.when(g >= NRED)
    def _():
        w_row = w_scr[pl.ds(g - NRED, 1), :]
        for e in range(E):
            out_ref[0, e] = jnp.broadcast_to(w_row[0, e], (H, W))


def kernel(x, W1, W2, b2):
    return pl.pallas_call(
        _body,
        grid=(NRED + B,),
        in_specs=[
            pl.BlockSpec(
                (1, CBLK, H, W),
                lambda g: (jnp.minimum(g // NJ, B - 1),
                           jnp.where(g < NRED, g - (g // NJ) * NJ, NJ - 1),
                           0, 0),
            ),
            pl.BlockSpec((C, RED), lambda g: (0, 0)),
            pl.BlockSpec((RED, E), lambda g: (0, 0)),
            pl.BlockSpec((1, E), lambda g: (0, 0)),
        ],
        out_specs=pl.BlockSpec(
            (1, E, H, W),
            lambda g: (jnp.maximum(g - NRED, 0), 0, 0, 0),
        ),
        out_shape=jax.ShapeDtypeStruct((B, E, H, W), jnp.float32),
        scratch_shapes=[
            pltpu.VMEM((1, C), jnp.float32),
            pltpu.VMEM((B, E), jnp.float32),
        ],
    )(x, W1.T, W2.T, b2.reshape(1, E))
